# Initial kernel scaffold; baseline (speedup 1.0000x reference)
import jax
import jax.numpy as jnp
from jax.experimental import pallas as pl


def kernel(z, scodebook, size):
    raise NotImplementedError("write your pallas kernel here")


# speedup vs baseline: 4.0436x; 4.0436x over previous
import functools

import jax
import jax.numpy as jnp
from jax.experimental import pallas as pl
from jax.experimental.pallas import tpu as pltpu

CODE_DIM = 256
CODEBOOK_NUM = 8
CODEBOOK_SIZE = 512
SCODEBOOK_ROWS = 1024
B = 8
T = 1024

_DIST_PREC = jax.lax.Precision.DEFAULT
_EXACT_PREC = jax.lax.Precision.HIGHEST


def _rvq_kernel(z_ref, scb_ref, sizei_ref, sizef_ref, zq_ref, mapped_ref,
                emb_scr, embsq_scr):
    b = pl.program_id(0)

    @pl.when(b == 0)
    def _init():
        size_col = sizei_ref[...]
        riota = jax.lax.broadcasted_iota(jnp.int32,
                                         (CODEBOOK_SIZE, SCODEBOOK_ROWS), 1)
        osel = (riota == size_col).astype(jnp.float32)
        emb = jax.lax.dot_general(
            osel, scb_ref[...], (((1,), (0,)), ((), ())),
            precision=_EXACT_PREC, preferred_element_type=jnp.float32)
        emb_scr[...] = emb
        embsq_scr[...] = jnp.sum(emb * emb, axis=1, keepdims=True)

    x = z_ref[0]
    emb = emb_scr[...]
    emb_sq = embsq_scr[...]
    sizef_row = sizef_ref[...]

    residual = x
    zq = jnp.zeros_like(x)
    mapped_rows = []
    for q in range(CODEBOOK_NUM):
        m = jax.lax.dot_general(
            emb, residual, (((1,), (0,)), ((), ())),
            precision=_DIST_PREC, preferred_element_type=jnp.float32)
        rsq = jnp.sum(residual * residual, axis=0, keepdims=True)
        d = (rsq - 2.0 * m) + emb_sq
        idx = jnp.argmin(d, axis=0)
        kiota = jax.lax.broadcasted_iota(jnp.int32, (CODEBOOK_SIZE, T), 0)
        onehot = (kiota == idx[None, :]).astype(jnp.float32)
        quant = jax.lax.dot_general(
            emb, onehot, (((0,), (0,)), ((), ())),
            precision=_EXACT_PREC, preferred_element_type=jnp.float32)
        zq = zq + quant
        residual = residual - quant
        mappedf = jax.lax.dot_general(
            sizef_row, onehot, (((1,), (0,)), ((), ())),
            precision=_EXACT_PREC, preferred_element_type=jnp.float32)
        mapped_rows.append(mappedf.astype(jnp.int32))

    mapped_ref[0] = jnp.concatenate(mapped_rows, axis=0)

    zq_ref[0] = x + (zq - x)


@functools.partial(jax.jit, static_argnames=())
def kernel(z, scodebook, size):
    sizei = size.reshape(CODEBOOK_SIZE, 1)
    sizef = size.astype(jnp.float32).reshape(1, CODEBOOK_SIZE)
    zq_bdt, mapped = pl.pallas_call(
        _rvq_kernel,
        grid=(B,),
        in_specs=[
            pl.BlockSpec((1, CODE_DIM, T), lambda b: (b, 0, 0)),
            pl.BlockSpec((SCODEBOOK_ROWS, CODE_DIM), lambda b: (0, 0)),
            pl.BlockSpec((CODEBOOK_SIZE, 1), lambda b: (0, 0)),
            pl.BlockSpec((1, CODEBOOK_SIZE), lambda b: (0, 0)),
        ],
        out_specs=[
            pl.BlockSpec((1, CODE_DIM, T), lambda b: (b, 0, 0)),
            pl.BlockSpec((1, CODEBOOK_NUM, T), lambda b: (b, 0, 0)),
        ],
        out_shape=[
            jax.ShapeDtypeStruct((B, CODE_DIM, T), jnp.float32),
            jax.ShapeDtypeStruct((B, CODEBOOK_NUM, T), jnp.int32),
        ],
        scratch_shapes=[
            pltpu.VMEM((CODEBOOK_SIZE, CODE_DIM), jnp.float32),
            pltpu.VMEM((CODEBOOK_SIZE, 1), jnp.float32),
        ],
    )(z, scodebook, sizei, sizef)
    zq = jnp.transpose(zq_bdt, (0, 2, 1))
    return zq, jnp.transpose(mapped, (1, 0, 2))


# speedup vs baseline: 7.0382x; 1.7406x over previous
import functools

import jax
import jax.numpy as jnp
from jax.experimental import pallas as pl
from jax.experimental.pallas import tpu as pltpu

CODE_DIM = 256
CODEBOOK_NUM = 8
CODEBOOK_SIZE = 512
SCODEBOOK_ROWS = 1024
B = 8
T = 1024

_DIST_PREC = jax.lax.Precision.DEFAULT
_EXACT_PREC = jax.lax.Precision.HIGHEST


def _rvq_kernel(z_ref, scb_ref, sizei_ref, zq_ref, mapped_ref,
                emb_scr, embsq_scr, embhi_scr, embmid_scr, emblo_scr):
    b = pl.program_id(0)

    @pl.when(b == 0)
    def _init():
        size_col = sizei_ref[...]
        riota = jax.lax.broadcasted_iota(jnp.int32,
                                         (CODEBOOK_SIZE, SCODEBOOK_ROWS), 1)
        osel = (riota == size_col).astype(jnp.float32)
        emb = jax.lax.dot_general(
            osel, scb_ref[...], (((1,), (0,)), ((), ())),
            precision=_EXACT_PREC, preferred_element_type=jnp.float32)
        emb_scr[...] = emb
        embsq_scr[...] = jnp.sum(emb * emb, axis=1, keepdims=True)
        hi = emb.astype(jnp.bfloat16)
        r1 = emb - hi.astype(jnp.float32)
        mid = r1.astype(jnp.bfloat16)
        lo = (r1 - mid.astype(jnp.float32)).astype(jnp.bfloat16)
        embhi_scr[...] = hi
        embmid_scr[...] = mid
        emblo_scr[...] = lo

    x = z_ref[0]
    emb = emb_scr[...]
    emb_sq = embsq_scr[...]
    emb_hi = embhi_scr[...]
    emb_mid = embmid_scr[...]
    emb_lo = emblo_scr[...]
    size_col = sizei_ref[...]

    residual = x
    zq = jnp.zeros_like(x)
    mapped_rows = []
    for q in range(CODEBOOK_NUM):
        m = jax.lax.dot_general(
            emb, residual, (((1,), (0,)), ((), ())),
            precision=_DIST_PREC, preferred_element_type=jnp.float32)
        rsq = jnp.sum(residual * residual, axis=0, keepdims=True)
        d = (rsq - 2.0 * m) + emb_sq
        idx = jnp.argmin(d, axis=0)
        kiota = jax.lax.broadcasted_iota(jnp.int32, (CODEBOOK_SIZE, T), 0)
        sel = kiota == idx[None, :]
        onehot = sel.astype(jnp.bfloat16)
        dn = (((0,), (0,)), ((), ()))
        quant = (jax.lax.dot_general(emb_hi, onehot, dn,
                                     preferred_element_type=jnp.float32)
                 + jax.lax.dot_general(emb_mid, onehot, dn,
                                       preferred_element_type=jnp.float32)
                 + jax.lax.dot_general(emb_lo, onehot, dn,
                                       preferred_element_type=jnp.float32))
        zq = zq + quant
        residual = residual - quant
        mapped_rows.append(jnp.sum(
            jnp.where(sel, size_col, 0), axis=0, keepdims=True))

    mapped_ref[0] = jnp.concatenate(mapped_rows, axis=0)

    zq_ref[0] = x + (zq - x)


@functools.partial(jax.jit, static_argnames=())
def kernel(z, scodebook, size):
    sizei = size.reshape(CODEBOOK_SIZE, 1)
    zq_bdt, mapped = pl.pallas_call(
        _rvq_kernel,
        grid=(B,),
        in_specs=[
            pl.BlockSpec((1, CODE_DIM, T), lambda b: (b, 0, 0)),
            pl.BlockSpec((SCODEBOOK_ROWS, CODE_DIM), lambda b: (0, 0)),
            pl.BlockSpec((CODEBOOK_SIZE, 1), lambda b: (0, 0)),
        ],
        out_specs=[
            pl.BlockSpec((1, CODE_DIM, T), lambda b: (b, 0, 0)),
            pl.BlockSpec((1, CODEBOOK_NUM, T), lambda b: (b, 0, 0)),
        ],
        out_shape=[
            jax.ShapeDtypeStruct((B, CODE_DIM, T), jnp.float32),
            jax.ShapeDtypeStruct((B, CODEBOOK_NUM, T), jnp.int32),
        ],
        scratch_shapes=[
            pltpu.VMEM((CODEBOOK_SIZE, CODE_DIM), jnp.float32),
            pltpu.VMEM((CODEBOOK_SIZE, 1), jnp.float32),
            pltpu.VMEM((CODEBOOK_SIZE, CODE_DIM), jnp.bfloat16),
            pltpu.VMEM((CODEBOOK_SIZE, CODE_DIM), jnp.bfloat16),
            pltpu.VMEM((CODEBOOK_SIZE, CODE_DIM), jnp.bfloat16),
        ],
    )(z, scodebook, sizei)
    zq = jnp.transpose(zq_bdt, (0, 2, 1))
    return zq, jnp.transpose(mapped, (1, 0, 2))
